# baseline (speedup 1.0000x reference)
import functools

import jax
import jax.numpy as jnp
from jax import lax
from jax.experimental import pallas as pl
from jax.experimental.pallas import tpu as pltpu
from jax.experimental.pallas import tpu_sc as plsc

N_NODES = 10000
N_EDGES = 320000
F = 16

NC, NS = 2, 16
NW = NC * NS
CHUNK = 2000
N_CHUNKS = 5
EDGES_PER_TILE = CHUNK * N_CHUNKS
N_PAD = 10240
BLK = N_PAD // 8
ROWS_PER_TILE = N_PAD // NS
NROW128 = N_PAD * F // 128


def _pi(n):
    q = jnp.floor(n.astype(jnp.float32) * (1.0 / BLK)).astype(jnp.int32)
    return (n - q * BLK) * 8 + q


def _mm1_body(x_ref, w_ref, e_ref, o_ref, p_ref):
    fw = jnp.dot(x_ref[...], w_ref[...], preferred_element_type=jnp.float32)
    fwp = jnp.concatenate(
        [fw, jnp.zeros((N_PAD - N_NODES, F), jnp.float32)], axis=0)
    for b in range(8):
        o_ref[:, b * F:(b + 1) * F] = fwp[b * BLK:(b + 1) * BLK, :]
    p_ref[...] = jnp.reshape(_pi(e_ref[...]), (2 * N_EDGES // 128, 128))


def _tc_matmul1(x, w, e3):
    return pl.pallas_call(
        _mm1_body,
        out_shape=(
            jax.ShapeDtypeStruct((BLK, 128), jnp.float32),
            jax.ShapeDtypeStruct((2 * N_EDGES // 128, 128), jnp.int32),
        ),
    )(x, w, e3)


def _mid_body(zp_ref, w_ref, o_ref):
    h = jnp.maximum(zp_ref[:NROW128, :] + zp_ref[NROW128:, :], 0.0)
    big = jnp.tile(w_ref[...], (8, 8))
    ri = lax.broadcasted_iota(jnp.int32, (128, 128), 0) // F
    ci = lax.broadcasted_iota(jnp.int32, (128, 128), 1) // F
    wbd = jnp.where(ri == ci, big, 0.0)
    o_ref[...] = jnp.dot(h, wbd, preferred_element_type=jnp.float32)


def _tc_relu_sum_matmul(zp128, w):
    return pl.pallas_call(
        _mid_body,
        out_shape=jax.ShapeDtypeStruct((NROW128, 128), jnp.float32),
    )(zp128, w)


def _sum_body(zp_ref, o_ref):
    s = zp_ref[:NROW128, :] + zp_ref[NROW128:, :]
    for b in range(7):
        o_ref[b * BLK:(b + 1) * BLK, :] = s[:, b * F:(b + 1) * F]
    o_ref[7 * BLK:N_NODES, :] = s[:N_NODES - 7 * BLK, 7 * F:8 * F]


def _tc_sum(zp128):
    return pl.pallas_call(
        _sum_body,
        out_shape=jax.ShapeDtypeStruct((N_NODES, F), jnp.float32),
    )(zp128)


def _sc_segsum_body(vals_hbm, pidx_hbm, zeros_hbm, out_hbm, *scratch):
    src_bufs = scratch[0:N_CHUNKS]
    dst_bufs = scratch[N_CHUNKS:2 * N_CHUNKS]
    rows = scratch[2 * N_CHUNKS:2 * N_CHUNKS + 2]
    acc_sh = scratch[2 * N_CHUNKS + 2]
    isem, gsem0, gsem1, ssem0, ssem1 = scratch[2 * N_CHUNKS + 3:]
    gsems = (gsem0, gsem1)
    ssems = (ssem0, ssem1)
    c = lax.axis_index("c")
    s = lax.axis_index("s")
    wid = c * NS + s
    with jax.named_scope("segsum_pre"):
        idx_cps = []
        for j in range(N_CHUNKS):
            base = wid * EDGES_PER_TILE + j * CHUNK
            idx_cps.append(pltpu.async_copy(
                pidx_hbm.at[0, pl.ds(base, CHUNK)], src_bufs[j], isem))
            idx_cps.append(pltpu.async_copy(
                pidx_hbm.at[1, pl.ds(base, CHUNK)], dst_bufs[j], isem))
        pltpu.sync_copy(zeros_hbm,
                        acc_sh.at[pl.ds(s * ROWS_PER_TILE, ROWS_PER_TILE)])
        for cp in idx_cps[:2]:
            cp.wait()
        plsc.subcore_barrier()
    with jax.named_scope("segsum_loop"):
        gcps = [pltpu.async_copy(vals_hbm.at[src_bufs[0]], rows[0], gsems[0])]
        scps = []
        for j in range(N_CHUNKS):
            if j + 1 < N_CHUNKS:
                for cp in idx_cps[2 * (j + 1):2 * (j + 2)]:
                    cp.wait()
                if j >= 1:
                    scps[j - 1].wait()
                gcps.append(pltpu.async_copy(
                    vals_hbm.at[src_bufs[j + 1]], rows[(j + 1) % 2],
                    gsems[(j + 1) % 2]))
            gcps[j].wait()
            scps.append(pltpu.async_copy(
                rows[j % 2], acc_sh.at[dst_bufs[j]], ssems[j % 2], add=True))
        scps[N_CHUNKS - 2].wait()
        scps[N_CHUNKS - 1].wait()
        plsc.subcore_barrier()
    with jax.named_scope("segsum_out"):
        pltpu.sync_copy(acc_sh.at[pl.ds(s * ROWS_PER_TILE, ROWS_PER_TILE)],
                        out_hbm.at[c, pl.ds(s * ROWS_PER_TILE, ROWS_PER_TILE)])


_sc_segsum = functools.partial(
    pl.kernel,
    mesh=plsc.VectorSubcoreMesh(core_axis_name="c", subcore_axis_name="s"),
    compiler_params=pltpu.CompilerParams(use_tc_tiling_on_sc=False),
    out_type=jax.ShapeDtypeStruct((NC, N_PAD, F), jnp.float32),
    scratch_types=(
        [pltpu.VMEM((CHUNK,), jnp.int32) for _ in range(2 * N_CHUNKS)]
        + [pltpu.VMEM((CHUNK, F), jnp.float32) for _ in range(2)]
        + [pltpu.VMEM_SHARED((N_PAD, F), jnp.float32)]
        + [pltpu.SemaphoreType.DMA] * 5
    ),
)(_sc_segsum_body)


def kernel(features, edge_index, W1, W2):
    zeros = jnp.zeros((ROWS_PER_TILE, F), jnp.float32)
    e3 = jnp.reshape(edge_index, (2, N_EDGES // 128, 128))
    fw128, pidx2 = _tc_matmul1(features, W1, e3)
    fw = jnp.reshape(fw128, (N_PAD, F))
    pidx = jnp.reshape(pidx2, (2, N_EDGES))
    z1p = _sc_segsum(fw, pidx, zeros)
    z1p128 = jnp.reshape(z1p, (NC * NROW128, 128))
    hw128 = _tc_relu_sum_matmul(z1p128, W2)
    hw = jnp.reshape(hw128, (N_PAD, F))
    z2p = _sc_segsum(hw, pidx, zeros)
    z2p128 = jnp.reshape(z2p, (NC * NROW128, 128))
    return _tc_sum(z2p128)


# speedup vs baseline: 1.0927x; 1.0927x over previous
import functools

import jax
import jax.numpy as jnp
from jax import lax
from jax.experimental import pallas as pl
from jax.experimental.pallas import tpu as pltpu
from jax.experimental.pallas import tpu_sc as plsc

N_NODES = 10000
N_EDGES = 320000
F = 16

NC, NS = 2, 16
NW = NC * NS
CHUNK = 2000
N_CHUNKS = 5
EDGES_PER_TILE = CHUNK * N_CHUNKS
N_PAD = 10240
BLK = N_PAD // 8
ROWS_PER_TILE = N_PAD // NS
NROW128 = N_PAD * F // 128


def _pi(n):
    q = jnp.floor(n.astype(jnp.float32) * (1.0 / BLK)).astype(jnp.int32)
    return (n - q * BLK) * 8 + q


def _mm1_body(x_ref, w_ref, e_ref, o_ref, p_ref):
    fw = jnp.dot(x_ref[...], w_ref[...], preferred_element_type=jnp.float32)
    fwp = jnp.concatenate(
        [fw, jnp.zeros((N_PAD - N_NODES, F), jnp.float32)], axis=0)
    for b in range(8):
        o_ref[:, b * F:(b + 1) * F] = fwp[b * BLK:(b + 1) * BLK, :]
    p_ref[...] = jnp.reshape(_pi(e_ref[...]), (2 * N_EDGES // 128, 128))


def _tc_matmul1(x, w, e3):
    return pl.pallas_call(
        _mm1_body,
        out_shape=(
            jax.ShapeDtypeStruct((BLK, 128), jnp.float32),
            jax.ShapeDtypeStruct((2 * N_EDGES // 128, 128), jnp.int32),
        ),
    )(x, w, e3)


def _mid_body(zp_ref, w_ref, o_ref):
    h = jnp.maximum(zp_ref[:NROW128, :] + zp_ref[NROW128:, :], 0.0)
    big = jnp.tile(w_ref[...], (8, 8))
    ri = lax.broadcasted_iota(jnp.int32, (128, 128), 0) // F
    ci = lax.broadcasted_iota(jnp.int32, (128, 128), 1) // F
    wbd = jnp.where(ri == ci, big, 0.0)
    o_ref[...] = jnp.dot(h, wbd, preferred_element_type=jnp.float32)


def _tc_relu_sum_matmul(zp128, w):
    return pl.pallas_call(
        _mid_body,
        out_shape=jax.ShapeDtypeStruct((NROW128, 128), jnp.float32),
    )(zp128, w)


def _sum_body(zp_ref, o_ref):
    s = zp_ref[:NROW128, :] + zp_ref[NROW128:, :]
    for b in range(7):
        o_ref[b * BLK:(b + 1) * BLK, :] = s[:, b * F:(b + 1) * F]
    o_ref[7 * BLK:N_NODES, :] = s[:N_NODES - 7 * BLK, 7 * F:8 * F]


def _tc_sum(zp128):
    return pl.pallas_call(
        _sum_body,
        out_shape=jax.ShapeDtypeStruct((N_NODES, F), jnp.float32),
    )(zp128)


def _sc_segsum_body(vals_hbm, pidx_hbm, zeros_hbm, out_hbm, *scratch):
    src_bufs = scratch[0:N_CHUNKS]
    dst_bufs = scratch[N_CHUNKS:2 * N_CHUNKS]
    rows = scratch[2 * N_CHUNKS:2 * N_CHUNKS + 3]
    acc_sh = scratch[2 * N_CHUNKS + 3]
    isem, gsem0, gsem1, ssem0, ssem1, zsem = scratch[2 * N_CHUNKS + 4:]
    gsems = (gsem0, gsem1)
    ssems = (ssem0, ssem1)
    c = lax.axis_index("c")
    s = lax.axis_index("s")
    wid = c * NS + s
    zcp = pltpu.async_copy(
        zeros_hbm, acc_sh.at[pl.ds(s * ROWS_PER_TILE, ROWS_PER_TILE)], zsem)
    idx_cps = []
    for j in range(N_CHUNKS):
        base = wid * EDGES_PER_TILE + j * CHUNK
        idx_cps.append(pltpu.async_copy(
            pidx_hbm.at[0, pl.ds(base, CHUNK)], src_bufs[j], isem))
        idx_cps.append(pltpu.async_copy(
            pidx_hbm.at[1, pl.ds(base, CHUNK)], dst_bufs[j], isem))
    idx_cps[0].wait()
    gcps = [pltpu.async_copy(vals_hbm.at[src_bufs[0]], rows[0], gsems[0])]
    idx_cps[2].wait()
    gcps.append(pltpu.async_copy(vals_hbm.at[src_bufs[1]], rows[1], gsems[1]))
    zcp.wait()
    plsc.subcore_barrier()
    scps = []
    for j in range(N_CHUNKS):
        gcps[j].wait()
        idx_cps[2 * j + 1].wait()
        scps.append(pltpu.async_copy(
            rows[j % 3], acc_sh.at[dst_bufs[j]], ssems[j % 2], add=True))
        if j + 2 < N_CHUNKS:
            idx_cps[2 * (j + 2)].wait()
            if j >= 1:
                scps[j - 1].wait()
            gcps.append(pltpu.async_copy(
                vals_hbm.at[src_bufs[j + 2]], rows[(j + 2) % 3],
                gsems[j % 2]))
    scps[N_CHUNKS - 2].wait()
    scps[N_CHUNKS - 1].wait()
    plsc.subcore_barrier()
    pltpu.sync_copy(acc_sh.at[pl.ds(s * ROWS_PER_TILE, ROWS_PER_TILE)],
                    out_hbm.at[c, pl.ds(s * ROWS_PER_TILE, ROWS_PER_TILE)])


_sc_segsum = functools.partial(
    pl.kernel,
    mesh=plsc.VectorSubcoreMesh(core_axis_name="c", subcore_axis_name="s"),
    compiler_params=pltpu.CompilerParams(use_tc_tiling_on_sc=False),
    out_type=jax.ShapeDtypeStruct((NC, N_PAD, F), jnp.float32),
    scratch_types=(
        [pltpu.VMEM((CHUNK,), jnp.int32) for _ in range(2 * N_CHUNKS)]
        + [pltpu.VMEM((CHUNK, F), jnp.float32) for _ in range(3)]
        + [pltpu.VMEM_SHARED((N_PAD, F), jnp.float32)]
        + [pltpu.SemaphoreType.DMA] * 6
    ),
)(_sc_segsum_body)


def kernel(features, edge_index, W1, W2):
    zeros = jnp.zeros((ROWS_PER_TILE, F), jnp.float32)
    e3 = jnp.reshape(edge_index, (2, N_EDGES // 128, 128))
    fw128, pidx2 = _tc_matmul1(features, W1, e3)
    fw = jnp.reshape(fw128, (N_PAD, F))
    pidx = jnp.reshape(pidx2, (2, N_EDGES))
    z1p = _sc_segsum(fw, pidx, zeros)
    z1p128 = jnp.reshape(z1p, (NC * NROW128, 128))
    hw128 = _tc_relu_sum_matmul(z1p128, W2)
    hw = jnp.reshape(hw128, (N_PAD, F))
    z2p = _sc_segsum(hw, pidx, zeros)
    z2p128 = jnp.reshape(z2p, (NC * NROW128, 128))
    return _tc_sum(z2p128)
